# baseline (speedup 1.0000x reference)
import dataclasses
import functools

import jax
import jax.numpy as jnp
from jax import lax
from jax.experimental import pallas as pl
from jax.experimental.pallas import tpu as pltpu
from jax.experimental.pallas import tpu_sc as plsc

N_ROWS = 100000
DIM = 64
N_BEH = 3
N_EDGES = 1000000

CHUNK = 25600
N_CHUNKS = 4
PAD_ROWS = CHUNK * N_CHUNKS
PAIR_ROWS = PAD_ROWS // 2
CHUNK_P = CHUNK // 2
ACC_P = CHUNK_P + 16
W = 64
N_WIN = N_EDGES // W
N_TILES = 16
WIN_PER_TILE = (N_WIN + N_TILES - 1) // N_TILES
ZROWS = ACC_P // N_TILES
FROWS = CHUNK_P // N_TILES


def _sc_segment_sums(user_pairs, item_pairs, edge_val, edge_u, edge_i):
    mesh = plsc.VectorSubcoreMesh(core_axis_name="c", subcore_axis_name="s")
    out_type = [
        jax.ShapeDtypeStruct((N_BEH, PAIR_ROWS, 128), jnp.float32),
        jax.ShapeDtypeStruct((N_BEH, PAIR_ROWS, 128), jnp.float32),
    ]
    cp = pltpu.CompilerParams()
    if "needs_layout_passes" in pltpu.CompilerParams.__dataclass_fields__:
        cp = dataclasses.replace(cp, needs_layout_passes=False)

    @functools.partial(
        pl.kernel,
        mesh=mesh,
        out_type=out_type,
        compiler_params=cp,
        scratch_types=[
            pltpu.VMEM((W,), jnp.int32),
            pltpu.VMEM((W,), jnp.int32),
            pltpu.VMEM((W,), jnp.int32),
            pltpu.VMEM((W,), jnp.int32),
            pltpu.VMEM((W,), jnp.int32),
            pltpu.VMEM((W,), jnp.int32),
            pltpu.VMEM((W,), jnp.float32),
            pltpu.VMEM((W, 128), jnp.float32),
            pltpu.VMEM((W, 128), jnp.float32),
            pltpu.VMEM((64, 128), jnp.float32),
            pltpu.VMEM_SHARED((ACC_P, 128), jnp.float32),
        ],
    )
    def seg_kernel(up_hbm, ip_hbm, ev_hbm, eu_hbm, ei_hbm, out_u, out_i,
                   didx, gidx, g2, pb, db, lidx, wv, rows, scaled, zbuf, acc):
        core = lax.axis_index("c")
        s = lax.axis_index("s")
        iota16 = jnp.arange(16, dtype=jnp.int32)
        zero16 = jnp.zeros((16,), jnp.float32)

        @pl.loop(0, 64)
        def _(r):
            for q in range(8):
                zbuf[r, pl.ds(16 * q, 16)] = zero16

        for b in range(N_BEH):
            for d in range(2):
                dest_hbm = eu_hbm if d == 0 else ei_hbm
                src_hbm = ei_hbm if d == 0 else eu_hbm
                table = ip_hbm if d == 0 else up_hbm
                out_hbm = out_u if d == 0 else out_i
                for cpass in range(2):
                    chunk = cpass * 2 + core
                    base = chunk * CHUNK
                    z0 = s * ZROWS
                    for t in range(ZROWS // 64):
                        pltpu.sync_copy(zbuf, acc.at[pl.ds(z0 + t * 64, 64)])
                    rem = ZROWS % 64
                    if rem:
                        pltpu.sync_copy(zbuf.at[pl.ds(0, rem)],
                                        acc.at[pl.ds(z0 + ZROWS - rem, rem)])
                    plsc.subcore_barrier()

                    trash = CHUNK + s * 2

                    def body(k, _):
                        win = s + k * N_TILES

                        @pl.when(win < N_WIN)
                        def _():
                            off = win * W
                            pltpu.sync_copy(dest_hbm.at[b, pl.ds(off, W)], didx)
                            pltpu.sync_copy(src_hbm.at[b, pl.ds(off, W)], gidx)
                            pltpu.sync_copy(ev_hbm.at[b, pl.ds(off, W)], wv)
                            for g in range(W // 16):
                                sl = pl.ds(16 * g, 16)
                                gv = gidx[sl]
                                g2[sl] = lax.shift_right_logical(gv, 1)
                                pb[sl] = lax.shift_left(gv & 1, 6)
                                dv = didx[sl]
                                loc = dv - base
                                inr = (loc >= 0) & (loc < CHUNK)
                                l = jnp.where(inr, loc, trash)
                                lidx[sl] = lax.shift_right_logical(l, 1)
                                db[sl] = lax.shift_left(l & 1, 6)
                            pltpu.sync_copy(table.at[g2], rows)

                            @pl.loop(0, W)
                            def _(j):
                                jj = jnp.full((16,), j, jnp.int32)
                                wspl = plsc.load_gather(wv, [jj])
                                cb = plsc.load_gather(pb, [jj])
                                dbs = plsc.load_gather(db, [jj])
                                for q in range(4):
                                    cq = iota16 + (16 * q)
                                    v = plsc.load_gather(rows, [jj, cb + cq])
                                    plsc.store_scatter(
                                        scaled, [jj, dbs + cq], v * wspl)
                                    plsc.store_scatter(
                                        scaled, [jj, (64 - dbs) + cq], zero16)

                            pltpu.sync_copy(scaled, acc.at[lidx], add=True)

                        return 0

                    lax.fori_loop(0, WIN_PER_TILE, body, 0)
                    plsc.subcore_barrier()
                    f0 = s * FROWS
                    pltpu.sync_copy(
                        acc.at[pl.ds(f0, FROWS)],
                        out_hbm.at[b, pl.ds(chunk * CHUNK_P + f0, FROWS)])
                    plsc.subcore_barrier()

    return seg_kernel(user_pairs, item_pairs, edge_val, edge_u, edge_i)


BLK = 2048


def _proj_body(x_ref, w_ref, stack_ref, mean_ref):
    acc = jnp.zeros((BLK, DIM), jnp.float32)
    for b in range(N_BEH):
        z = jnp.dot(x_ref[b], w_ref[...], preferred_element_type=jnp.float32)
        stack_ref[b] = jax.nn.sigmoid(z)
        acc = acc + z
    mean_ref[...] = jax.nn.sigmoid(acc * (1.0 / N_BEH))


def _project(stack, weight):
    grid = (PAD_ROWS // BLK,)
    return pl.pallas_call(
        _proj_body,
        grid=grid,
        in_specs=[
            pl.BlockSpec((N_BEH, BLK, DIM), lambda i: (0, i, 0)),
            pl.BlockSpec((DIM, DIM), lambda i: (0, 0)),
        ],
        out_specs=[
            pl.BlockSpec((N_BEH, BLK, DIM), lambda i: (0, i, 0)),
            pl.BlockSpec((BLK, DIM), lambda i: (i, 0)),
        ],
        out_shape=[
            jax.ShapeDtypeStruct((N_BEH, PAD_ROWS, DIM), jnp.float32),
            jax.ShapeDtypeStruct((PAD_ROWS, DIM), jnp.float32),
        ],
    )(stack, weight)


def kernel(user_embedding, item_embedding, u_w, i_w, edge_val, edge_u, edge_i):
    user_pairs = user_embedding.reshape(N_ROWS // 2, 2 * DIM)
    item_pairs = item_embedding.reshape(N_ROWS // 2, 2 * DIM)
    ue_p, ie_p = _sc_segment_sums(
        user_pairs, item_pairs, edge_val, edge_u, edge_i)
    ue_stack = ue_p.reshape(N_BEH, PAD_ROWS, DIM)
    ie_stack = ie_p.reshape(N_BEH, PAD_ROWS, DIM)
    us_out, u_mean = _project(ue_stack, u_w)
    is_out, i_mean = _project(ie_stack, i_w)
    return (
        u_mean[:N_ROWS],
        i_mean[:N_ROWS],
        us_out[:, :N_ROWS],
        is_out[:, :N_ROWS],
    )


# speedup vs baseline: 2.2911x; 2.2911x over previous
import dataclasses
import functools

import jax
import jax.numpy as jnp
from jax import lax
from jax.experimental import pallas as pl
from jax.experimental.pallas import tpu as pltpu
from jax.experimental.pallas import tpu_sc as plsc

N_ROWS = 100000
DIM = 64
N_BEH = 3
N_EDGES = 1000000

CHUNK = 25600
N_CHUNKS = 4
PAD_ROWS = CHUNK * N_CHUNKS
PAIR_ROWS = PAD_ROWS // 2
CHUNK_P = CHUNK // 2
ACC_P = CHUNK_P + 128
W = 64
N_WIN = N_EDGES // W
N_TILES = 16
T_WIN = (N_WIN + N_TILES - 1) // N_TILES
NG = (T_WIN + 1 + 5) // 6
ZROWS = ACC_P // N_TILES
FROWS = CHUNK_P // N_TILES


def _sc_segment_sums(user_dup, item_dup, zeros, edge_val, edge_u, edge_i):
    mesh = plsc.VectorSubcoreMesh(core_axis_name="c", subcore_axis_name="s")
    out_type = [
        jax.ShapeDtypeStruct((N_BEH, PAIR_ROWS, 128), jnp.float32),
        jax.ShapeDtypeStruct((N_BEH, PAIR_ROWS, 128), jnp.float32),
    ]
    cp = pltpu.CompilerParams()
    if "needs_layout_passes" in pltpu.CompilerParams.__dataclass_fields__:
        cp = dataclasses.replace(cp, needs_layout_passes=False)

    @functools.partial(
        pl.kernel,
        mesh=mesh,
        out_type=out_type,
        compiler_params=cp,
        scratch_types=(
            [pltpu.VMEM((W,), jnp.int32) for _ in range(3)]
            + [pltpu.VMEM((W,), jnp.int32) for _ in range(3)]
            + [pltpu.VMEM((W,), jnp.float32) for _ in range(3)]
            + [pltpu.VMEM((W,), jnp.int32) for _ in range(2)]
            + [pltpu.VMEM((W,), jnp.int32) for _ in range(2)]
            + [pltpu.VMEM((W, 128), jnp.float32) for _ in range(2)]
            + [pltpu.VMEM((W, 128), jnp.float32)]
            + [pltpu.VMEM_SHARED((ACC_P, 128), jnp.float32)]
            + [pltpu.SemaphoreType.DMA for _ in range(3)]
            + [pltpu.SemaphoreType.DMA for _ in range(2)]
        ),
    )
    def seg_kernel(ud_hbm, id_hbm, z_hbm, ev_hbm, eu_hbm, ei_hbm, out_u, out_i,
                   d0, d1, d2, g0, g1, g2, w0, w1, w2, l0, l1, h0, h1,
                   r0, r1, scaled, acc, si0, si1, si2, sr0, sr1):
        core = lax.axis_index("c")
        s = lax.axis_index("s")
        zero16 = jnp.zeros((16,), jnp.float32)
        I_d, I_g, I_w = (d0, d1, d2), (g0, g1, g2), (w0, w1, w2)
        M_l, M_h = (l0, l1), (h0, h1)
        R = (r0, r1)
        S_i = (si0, si1, si2)
        S_r = (sr0, sr1)

        for d in range(2):
            dest_hbm = eu_hbm if d == 0 else ei_hbm
            src_hbm = ei_hbm if d == 0 else eu_hbm
            table = id_hbm if d == 0 else ud_hbm
            out_hbm = out_u if d == 0 else out_i

            def pass_body(q, _, dest_hbm=dest_hbm, src_hbm=src_hbm,
                          table=table, out_hbm=out_hbm):
                b = q // 2
                cpass = q % 2
                chunk = cpass * 2 + core
                base = chunk * CHUNK
                trash = CHUNK + s * 2

                z0 = s * ZROWS
                pltpu.sync_copy(z_hbm.at[pl.ds(z0, ZROWS)],
                                acc.at[pl.ds(z0, ZROWS)])
                plsc.subcore_barrier()

                def idx_copies(n, slot):
                    off = (s + n * N_TILES) * W
                    return (
                        pltpu.make_async_copy(
                            dest_hbm.at[b, pl.ds(off, W)], I_d[slot], S_i[slot]),
                        pltpu.make_async_copy(
                            src_hbm.at[b, pl.ds(off, W)], I_g[slot], S_i[slot]),
                        pltpu.make_async_copy(
                            ev_hbm.at[b, pl.ds(off, W)], I_w[slot], S_i[slot]),
                    )

                def gather_copy(slot3, slot2):
                    return pltpu.make_async_copy(
                        table.at[I_g[slot3]], R[slot2], S_r[slot2])

                for c in idx_copies(0, 0):
                    c.start()

                @pl.loop(0, NG)
                def _(gg):
                    for p in range(6):
                        n = gg * 6 + p
                        i3, nxt3, prev3 = p % 3, (p + 1) % 3, (p + 2) % 3
                        r2, prev2 = p % 2, (p + 1) % 2
                        w_n = s + n * N_TILES

                        @pl.when((n < T_WIN - 1)
                                 & (w_n + N_TILES < N_WIN))
                        def _():
                            for c in idx_copies(n + 1, nxt3):
                                c.start()

                        @pl.when((n < T_WIN) & (w_n < N_WIN))
                        def _():
                            for c in idx_copies(n, i3):
                                c.wait()
                            gather_copy(i3, r2).start()
                            for g in range(W // 16):
                                sl = pl.ds(16 * g, 16)
                                dv = I_d[i3][sl]
                                loc = dv - base
                                inr = (loc >= 0) & (loc < CHUNK)
                                l = jnp.where(inr, loc, trash)
                                M_l[r2][sl] = lax.shift_right_logical(l, 1)
                                M_h[r2][sl] = lax.shift_left(l & 1, 6)

                        @pl.when((n >= 1) & (w_n - N_TILES < N_WIN))
                        def _():
                            gather_copy(prev3, prev2).wait()

                            @pl.loop(0, W)
                            def _(j):
                                jj = jnp.full((16,), j, jnp.int32)
                                wspl = plsc.load_gather(I_w[prev3], [jj])
                                dbs = plsc.load_gather(M_h[prev2], [jj])
                                f0 = jnp.where(dbs == 0, wspl, zero16)
                                f1 = wspl - f0
                                for qq in range(4):
                                    v = R[prev2][j, pl.ds(16 * qq, 16)]
                                    scaled[j, pl.ds(16 * qq, 16)] = v * f0
                                    scaled[j, pl.ds(64 + 16 * qq, 16)] = v * f1

                            pltpu.sync_copy(scaled, acc.at[M_l[prev2]],
                                            add=True)

                plsc.subcore_barrier()
                f0r = s * FROWS
                pltpu.sync_copy(
                    acc.at[pl.ds(f0r, FROWS)],
                    out_hbm.at[b, pl.ds(chunk * CHUNK_P + f0r, FROWS)])
                plsc.subcore_barrier()
                return 0

            lax.fori_loop(0, N_BEH * 2, pass_body, 0)

    return seg_kernel(user_dup, item_dup, zeros, edge_val, edge_u, edge_i)


BLK = 2048


def _proj_body(x_ref, w_ref, stack_ref, mean_ref):
    acc = jnp.zeros((BLK, DIM), jnp.float32)
    for b in range(N_BEH):
        z = jnp.dot(x_ref[b], w_ref[...], preferred_element_type=jnp.float32)
        stack_ref[b] = jax.nn.sigmoid(z)
        acc = acc + z
    mean_ref[...] = jax.nn.sigmoid(acc * (1.0 / N_BEH))


def _project(stack, weight):
    grid = (PAD_ROWS // BLK,)
    return pl.pallas_call(
        _proj_body,
        grid=grid,
        in_specs=[
            pl.BlockSpec((N_BEH, BLK, DIM), lambda i: (0, i, 0)),
            pl.BlockSpec((DIM, DIM), lambda i: (0, 0)),
        ],
        out_specs=[
            pl.BlockSpec((N_BEH, BLK, DIM), lambda i: (0, i, 0)),
            pl.BlockSpec((BLK, DIM), lambda i: (i, 0)),
        ],
        out_shape=[
            jax.ShapeDtypeStruct((N_BEH, PAD_ROWS, DIM), jnp.float32),
            jax.ShapeDtypeStruct((PAD_ROWS, DIM), jnp.float32),
        ],
    )(stack, weight)


def kernel(user_embedding, item_embedding, u_w, i_w, edge_val, edge_u, edge_i):
    user_dup = jnp.concatenate([user_embedding, user_embedding], axis=1)
    item_dup = jnp.concatenate([item_embedding, item_embedding], axis=1)
    zeros = jnp.zeros((ACC_P, 128), jnp.float32)
    ue_p, ie_p = _sc_segment_sums(
        user_dup, item_dup, zeros, edge_val, edge_u, edge_i)
    ue_stack = ue_p.reshape(N_BEH, PAD_ROWS, DIM)
    ie_stack = ie_p.reshape(N_BEH, PAD_ROWS, DIM)
    us_out, u_mean = _project(ue_stack, u_w)
    is_out, i_mean = _project(ie_stack, i_w)
    return (
        u_mean[:N_ROWS],
        i_mean[:N_ROWS],
        us_out[:, :N_ROWS],
        is_out[:, :N_ROWS],
    )


# speedup vs baseline: 2.3743x; 1.0363x over previous
import dataclasses
import functools

import jax
import jax.numpy as jnp
from jax import lax
from jax.experimental import pallas as pl
from jax.experimental.pallas import tpu as pltpu
from jax.experimental.pallas import tpu_sc as plsc

N_ROWS = 100000
DIM = 64
N_BEH = 3
N_EDGES = 1000000

CHUNK = 25600
N_CHUNKS = 4
PAD_ROWS = CHUNK * N_CHUNKS
PAIR_ROWS = PAD_ROWS // 2
CHUNK_P = CHUNK // 2
ACC_P = CHUNK_P + 128
W = 64
N_WIN = N_EDGES // W
N_TILES = 16
T_WIN = (N_WIN + N_TILES - 1) // N_TILES
NG = (T_WIN + 1 + 5) // 6
ZROWS = ACC_P // N_TILES
FROWS = CHUNK_P // N_TILES


def _sc_segment_sums(user_dup, item_dup, zeros, edge_val, edge_u, edge_i):
    mesh = plsc.VectorSubcoreMesh(core_axis_name="c", subcore_axis_name="s")
    out_type = [
        jax.ShapeDtypeStruct((N_BEH, PAIR_ROWS, 128), jnp.float32),
        jax.ShapeDtypeStruct((N_BEH, PAIR_ROWS, 128), jnp.float32),
    ]
    cp = pltpu.CompilerParams()
    if "needs_layout_passes" in pltpu.CompilerParams.__dataclass_fields__:
        cp = dataclasses.replace(cp, needs_layout_passes=False)

    @functools.partial(
        pl.kernel,
        mesh=mesh,
        out_type=out_type,
        compiler_params=cp,
        scratch_types=(
            [pltpu.VMEM((W,), jnp.int32) for _ in range(3)]
            + [pltpu.VMEM((W,), jnp.int32) for _ in range(3)]
            + [pltpu.VMEM((W,), jnp.float32) for _ in range(3)]
            + [pltpu.VMEM((W,), jnp.int32) for _ in range(2)]
            + [pltpu.VMEM((W,), jnp.int32) for _ in range(2)]
            + [pltpu.VMEM((W, 128), jnp.float32) for _ in range(2)]
            + [pltpu.VMEM((W, 128), jnp.float32)]
            + [pltpu.VMEM_SHARED((ACC_P, 128), jnp.float32)]
            + [pltpu.SemaphoreType.DMA for _ in range(3)]
            + [pltpu.SemaphoreType.DMA for _ in range(2)]
        ),
    )
    def seg_kernel(ud_hbm, id_hbm, z_hbm, ev_hbm, eu_hbm, ei_hbm, out_u, out_i,
                   d0, d1, d2, g0, g1, g2, w0, w1, w2, l0, l1, h0, h1,
                   r0, r1, scaled, acc, si0, si1, si2, sr0, sr1):
        core = lax.axis_index("c")
        s = lax.axis_index("s")
        zero16 = jnp.zeros((16,), jnp.float32)
        I_d, I_g, I_w = (d0, d1, d2), (g0, g1, g2), (w0, w1, w2)
        M_l, M_h = (l0, l1), (h0, h1)
        R = (r0, r1)
        S_i = (si0, si1, si2)
        S_r = (sr0, sr1)

        for d in range(2):
            dest_hbm = eu_hbm if d == 0 else ei_hbm
            src_hbm = ei_hbm if d == 0 else eu_hbm
            table = id_hbm if d == 0 else ud_hbm
            out_hbm = out_u if d == 0 else out_i

            def pass_body(q, _, dest_hbm=dest_hbm, src_hbm=src_hbm,
                          table=table, out_hbm=out_hbm):
                b = q // 2
                cpass = q % 2
                chunk = cpass * 2 + core
                base = chunk * CHUNK
                trash = CHUNK + s * 2

                z0 = s * ZROWS
                pltpu.sync_copy(z_hbm.at[pl.ds(z0, ZROWS)],
                                acc.at[pl.ds(z0, ZROWS)])
                plsc.subcore_barrier()

                def idx_copies(n, slot):
                    off = (s + n * N_TILES) * W
                    return (
                        pltpu.make_async_copy(
                            dest_hbm.at[b, pl.ds(off, W)], I_d[slot], S_i[slot]),
                        pltpu.make_async_copy(
                            src_hbm.at[b, pl.ds(off, W)], I_g[slot], S_i[slot]),
                        pltpu.make_async_copy(
                            ev_hbm.at[b, pl.ds(off, W)], I_w[slot], S_i[slot]),
                    )

                def gather_copy(slot3, slot2):
                    return pltpu.make_async_copy(
                        table.at[I_g[slot3]], R[slot2], S_r[slot2])

                for c in idx_copies(0, 0):
                    c.start()

                @pl.loop(0, NG)
                def _(gg):
                    for p in range(6):
                        n = gg * 6 + p
                        i3, nxt3, prev3 = p % 3, (p + 1) % 3, (p + 2) % 3
                        r2, prev2 = p % 2, (p + 1) % 2
                        w_n = s + n * N_TILES

                        @pl.when((n < T_WIN - 1)
                                 & (w_n + N_TILES < N_WIN))
                        def _():
                            for c in idx_copies(n + 1, nxt3):
                                c.start()

                        @pl.when((n < T_WIN) & (w_n < N_WIN))
                        def _():
                            for c in idx_copies(n, i3):
                                c.wait()
                            gather_copy(i3, r2).start()
                            for g in range(W // 16):
                                sl = pl.ds(16 * g, 16)
                                dv = I_d[i3][sl]
                                loc = dv - base
                                inr = (loc >= 0) & (loc < CHUNK)
                                l = jnp.where(inr, loc, trash)
                                M_l[r2][sl] = lax.shift_right_logical(l, 1)
                                M_h[r2][sl] = lax.shift_left(l & 1, 6)

                        @pl.when((n >= 1) & (w_n - N_TILES < N_WIN))
                        def _():
                            gather_copy(prev3, prev2).wait()

                            @pl.loop(0, W, step=4)
                            def _(j0):
                                for u in range(4):
                                    j = j0 + u
                                    jj = jnp.full((16,), j, jnp.int32)
                                    wspl = plsc.load_gather(I_w[prev3], [jj])
                                    dbs = plsc.load_gather(M_h[prev2], [jj])
                                    f0 = jnp.where(dbs == 0, wspl, zero16)
                                    f1 = wspl - f0
                                    for qq in range(4):
                                        v = R[prev2][j, pl.ds(16 * qq, 16)]
                                        scaled[j, pl.ds(16 * qq, 16)] = v * f0
                                        scaled[j, pl.ds(64 + 16 * qq, 16)] = (
                                            v * f1)

                            pltpu.sync_copy(scaled, acc.at[M_l[prev2]],
                                            add=True)

                plsc.subcore_barrier()
                f0r = s * FROWS
                pltpu.sync_copy(
                    acc.at[pl.ds(f0r, FROWS)],
                    out_hbm.at[b, pl.ds(chunk * CHUNK_P + f0r, FROWS)])
                plsc.subcore_barrier()
                return 0

            lax.fori_loop(0, N_BEH * 2, pass_body, 0)

    return seg_kernel(user_dup, item_dup, zeros, edge_val, edge_u, edge_i)


BLK = 2048


def _proj_body(x_ref, w_ref, stack_ref, mean_ref):
    acc = jnp.zeros((BLK, DIM), jnp.float32)
    for b in range(N_BEH):
        z = jnp.dot(x_ref[b], w_ref[...], preferred_element_type=jnp.float32)
        stack_ref[b] = jax.nn.sigmoid(z)
        acc = acc + z
    mean_ref[...] = jax.nn.sigmoid(acc * (1.0 / N_BEH))


def _project(stack, weight):
    grid = (PAD_ROWS // BLK,)
    return pl.pallas_call(
        _proj_body,
        grid=grid,
        in_specs=[
            pl.BlockSpec((N_BEH, BLK, DIM), lambda i: (0, i, 0)),
            pl.BlockSpec((DIM, DIM), lambda i: (0, 0)),
        ],
        out_specs=[
            pl.BlockSpec((N_BEH, BLK, DIM), lambda i: (0, i, 0)),
            pl.BlockSpec((BLK, DIM), lambda i: (i, 0)),
        ],
        out_shape=[
            jax.ShapeDtypeStruct((N_BEH, PAD_ROWS, DIM), jnp.float32),
            jax.ShapeDtypeStruct((PAD_ROWS, DIM), jnp.float32),
        ],
    )(stack, weight)


def kernel(user_embedding, item_embedding, u_w, i_w, edge_val, edge_u, edge_i):
    user_dup = jnp.concatenate([user_embedding, user_embedding], axis=1)
    item_dup = jnp.concatenate([item_embedding, item_embedding], axis=1)
    zeros = jnp.zeros((ACC_P, 128), jnp.float32)
    ue_p, ie_p = _sc_segment_sums(
        user_dup, item_dup, zeros, edge_val, edge_u, edge_i)
    ue_stack = ue_p.reshape(N_BEH, PAD_ROWS, DIM)
    ie_stack = ie_p.reshape(N_BEH, PAD_ROWS, DIM)
    us_out, u_mean = _project(ue_stack, u_w)
    is_out, i_mean = _project(ie_stack, i_w)
    return (
        u_mean[:N_ROWS],
        i_mean[:N_ROWS],
        us_out[:, :N_ROWS],
        is_out[:, :N_ROWS],
    )


# speedup vs baseline: 2.4657x; 1.0385x over previous
import dataclasses
import functools

import jax
import jax.numpy as jnp
from jax import lax
from jax.experimental import pallas as pl
from jax.experimental.pallas import tpu as pltpu
from jax.experimental.pallas import tpu_sc as plsc

N_ROWS = 100000
DIM = 64
N_BEH = 3
N_EDGES = 1000000

CHUNK = 25600
N_CHUNKS = 4
PAD_ROWS = CHUNK * N_CHUNKS
PAIR_ROWS = PAD_ROWS // 2
CHUNK_P = CHUNK // 2
ACC_P = CHUNK_P + 128
W = 64
N_WIN = N_EDGES // W
N_TILES = 16
T_WIN = (N_WIN + N_TILES - 1) // N_TILES
NG = (T_WIN + 1 + 5) // 6
ZROWS = ACC_P // N_TILES
FROWS = CHUNK_P // N_TILES


def _sc_segment_sums(user_dup, item_dup, zeros, edge_val, edge_u, edge_i):
    mesh = plsc.VectorSubcoreMesh(core_axis_name="c", subcore_axis_name="s")
    out_type = [
        jax.ShapeDtypeStruct((N_BEH, PAIR_ROWS, 128), jnp.float32),
        jax.ShapeDtypeStruct((N_BEH, PAIR_ROWS, 128), jnp.float32),
    ]
    cp = pltpu.CompilerParams()
    if "needs_layout_passes" in pltpu.CompilerParams.__dataclass_fields__:
        cp = dataclasses.replace(cp, needs_layout_passes=False)

    @functools.partial(
        pl.kernel,
        mesh=mesh,
        out_type=out_type,
        compiler_params=cp,
        scratch_types=(
            [pltpu.VMEM((W,), jnp.int32) for _ in range(3)]
            + [pltpu.VMEM((W,), jnp.int32) for _ in range(3)]
            + [pltpu.VMEM((W,), jnp.float32) for _ in range(3)]
            + [pltpu.VMEM((W // 2,), jnp.int32) for _ in range(4)]
            + [pltpu.VMEM((W,), jnp.int32) for _ in range(2)]
            + [pltpu.VMEM((W, 128), jnp.float32) for _ in range(2)]
            + [pltpu.VMEM((W // 2, 128), jnp.float32) for _ in range(2)]
            + [pltpu.VMEM_SHARED((ACC_P, 128), jnp.float32)]
            + [pltpu.SemaphoreType.DMA for _ in range(3)]
            + [pltpu.SemaphoreType.DMA for _ in range(2)]
            + [pltpu.SemaphoreType.DMA for _ in range(2)]
        ),
    )
    def seg_kernel(ud_hbm, id_hbm, z_hbm, ev_hbm, eu_hbm, ei_hbm, out_u, out_i,
                   d0, d1, d2, g0, g1, g2, w0, w1, w2, la0, la1, lb0, lb1,
                   h0, h1, r0, r1, sca, scb, acc, si0, si1, si2, sr0, sr1,
                   ssa, ssb):
        core = lax.axis_index("c")
        s = lax.axis_index("s")
        zero16 = jnp.zeros((16,), jnp.float32)
        I_d, I_g, I_w = (d0, d1, d2), (g0, g1, g2), (w0, w1, w2)
        M_la, M_lb, M_h = (la0, la1), (lb0, lb1), (h0, h1)
        R = (r0, r1)
        S_i = (si0, si1, si2)
        S_r = (sr0, sr1)

        for d in range(2):
            dest_hbm = eu_hbm if d == 0 else ei_hbm
            src_hbm = ei_hbm if d == 0 else eu_hbm
            table = id_hbm if d == 0 else ud_hbm
            out_hbm = out_u if d == 0 else out_i

            def pass_body(q, _, dest_hbm=dest_hbm, src_hbm=src_hbm,
                          table=table, out_hbm=out_hbm):
                b = q // 2
                cpass = q % 2
                chunk = cpass * 2 + core
                base = chunk * CHUNK
                trash = CHUNK + s * 2

                z0 = s * ZROWS
                pltpu.sync_copy(z_hbm.at[pl.ds(z0, ZROWS)],
                                acc.at[pl.ds(z0, ZROWS)])
                plsc.subcore_barrier()

                def idx_copies(n, slot):
                    off = (s + n * N_TILES) * W
                    return (
                        pltpu.make_async_copy(
                            dest_hbm.at[b, pl.ds(off, W)], I_d[slot], S_i[slot]),
                        pltpu.make_async_copy(
                            src_hbm.at[b, pl.ds(off, W)], I_g[slot], S_i[slot]),
                        pltpu.make_async_copy(
                            ev_hbm.at[b, pl.ds(off, W)], I_w[slot], S_i[slot]),
                    )

                def gather_copy(slot3, slot2):
                    return pltpu.make_async_copy(
                        table.at[I_g[slot3]], R[slot2], S_r[slot2])

                for c in idx_copies(0, 0):
                    c.start()

                @pl.loop(0, NG)
                def _(gg):
                    for p in range(6):
                        n = gg * 6 + p
                        i3, nxt3, prev3 = p % 3, (p + 1) % 3, (p + 2) % 3
                        r2, prev2 = p % 2, (p + 1) % 2
                        w_n = s + n * N_TILES

                        @pl.when((n < T_WIN - 1)
                                 & (w_n + N_TILES < N_WIN))
                        def _():
                            for c in idx_copies(n + 1, nxt3):
                                c.start()

                        @pl.when((n < T_WIN) & (w_n < N_WIN))
                        def _():
                            @pl.when(n >= 2)
                            def _():
                                pltpu.make_async_copy(
                                    sca, acc.at[M_la[r2]], ssa).wait()
                                pltpu.make_async_copy(
                                    scb, acc.at[M_lb[r2]], ssb).wait()

                            for c in idx_copies(n, i3):
                                c.wait()
                            gather_copy(i3, r2).start()
                            for g in range(W // 16):
                                sl = pl.ds(16 * g, 16)
                                dv = I_d[i3][sl]
                                loc = dv - base
                                inr = (loc >= 0) & (loc < CHUNK)
                                l = jnp.where(inr, loc, trash)
                                lp = lax.shift_right_logical(l, 1)
                                if g < 2:
                                    M_la[r2][sl] = lp
                                else:
                                    M_lb[r2][pl.ds(16 * (g - 2), 16)] = lp
                                M_h[r2][sl] = lax.shift_left(l & 1, 6)

                        @pl.when((n >= 1) & (w_n - N_TILES < N_WIN))
                        def _():
                            gather_copy(prev3, prev2).wait()

                            def scale_rows(lo, dst):
                                @pl.loop(lo, lo + W // 2, step=4)
                                def _(j0):
                                    for u in range(4):
                                        j = j0 + u
                                        jj = jnp.full((16,), j, jnp.int32)
                                        wspl = plsc.load_gather(
                                            I_w[prev3], [jj])
                                        dbs = plsc.load_gather(
                                            M_h[prev2], [jj])
                                        f0 = jnp.where(dbs == 0, wspl, zero16)
                                        f1 = wspl - f0
                                        for qq in range(4):
                                            v = R[prev2][j, pl.ds(16 * qq, 16)]
                                            dj = j - lo
                                            dst[dj, pl.ds(16 * qq, 16)] = (
                                                v * f0)
                                            dst[dj, pl.ds(64 + 16 * qq, 16)] = (
                                                v * f1)

                            scale_rows(0, sca)
                            pltpu.make_async_copy(
                                sca, acc.at[M_la[prev2]], ssa).start(add=True)
                            scale_rows(W // 2, scb)
                            pltpu.make_async_copy(
                                scb, acc.at[M_lb[prev2]], ssb).start(add=True)

                for _ in range(2):
                    pltpu.make_async_copy(sca, acc.at[M_la[0]], ssa).wait()
                    pltpu.make_async_copy(scb, acc.at[M_lb[0]], ssb).wait()

                plsc.subcore_barrier()
                f0r = s * FROWS
                pltpu.sync_copy(
                    acc.at[pl.ds(f0r, FROWS)],
                    out_hbm.at[b, pl.ds(chunk * CHUNK_P + f0r, FROWS)])
                plsc.subcore_barrier()
                return 0

            lax.fori_loop(0, N_BEH * 2, pass_body, 0)

    return seg_kernel(user_dup, item_dup, zeros, edge_val, edge_u, edge_i)


BLK = 2048


def _proj_body(x_ref, w_ref, stack_ref, mean_ref):
    acc = jnp.zeros((BLK, DIM), jnp.float32)
    for b in range(N_BEH):
        z = jnp.dot(x_ref[b], w_ref[...], preferred_element_type=jnp.float32)
        stack_ref[b] = jax.nn.sigmoid(z)
        acc = acc + z
    mean_ref[...] = jax.nn.sigmoid(acc * (1.0 / N_BEH))


def _project(stack, weight):
    grid = (PAD_ROWS // BLK,)
    return pl.pallas_call(
        _proj_body,
        grid=grid,
        in_specs=[
            pl.BlockSpec((N_BEH, BLK, DIM), lambda i: (0, i, 0)),
            pl.BlockSpec((DIM, DIM), lambda i: (0, 0)),
        ],
        out_specs=[
            pl.BlockSpec((N_BEH, BLK, DIM), lambda i: (0, i, 0)),
            pl.BlockSpec((BLK, DIM), lambda i: (i, 0)),
        ],
        out_shape=[
            jax.ShapeDtypeStruct((N_BEH, PAD_ROWS, DIM), jnp.float32),
            jax.ShapeDtypeStruct((PAD_ROWS, DIM), jnp.float32),
        ],
    )(stack, weight)


def kernel(user_embedding, item_embedding, u_w, i_w, edge_val, edge_u, edge_i):
    user_dup = jnp.concatenate([user_embedding, user_embedding], axis=1)
    item_dup = jnp.concatenate([item_embedding, item_embedding], axis=1)
    zeros = jnp.zeros((ACC_P, 128), jnp.float32)
    ue_p, ie_p = _sc_segment_sums(
        user_dup, item_dup, zeros, edge_val, edge_u, edge_i)
    ue_stack = ue_p.reshape(N_BEH, PAD_ROWS, DIM)
    ie_stack = ie_p.reshape(N_BEH, PAD_ROWS, DIM)
    us_out, u_mean = _project(ue_stack, u_w)
    is_out, i_mean = _project(ie_stack, i_w)
    return (
        u_mean[:N_ROWS],
        i_mean[:N_ROWS],
        us_out[:, :N_ROWS],
        is_out[:, :N_ROWS],
    )


# speedup vs baseline: 6.7727x; 2.7467x over previous
import dataclasses
import functools

import jax
import jax.numpy as jnp
from jax import lax
from jax.experimental import pallas as pl
from jax.experimental.pallas import tpu as pltpu
from jax.experimental.pallas import tpu_sc as plsc

N_ROWS = 100000
DIM = 64
N_BEH = 3
N_EDGES = 1000000

CHUNK = 25600
N_CHUNKS = 4
PAD_ROWS = CHUNK * N_CHUNKS
PAIR_ROWS = PAD_ROWS // 2
CHUNK_P = CHUNK // 2
ACC_P = CHUNK_P + 128
W = 64
N_WIN = N_EDGES // W
N_TILES = 16
ZROWS = ACC_P // N_TILES
FROWS = CHUNK_P // N_TILES

CAP = 8960
NREG = 2 * N_BEH * 32 * N_CHUNKS
K1 = (N_WIN + 31) // 32
K1H = (K1 + 1) // 2
T2 = 2 * CAP // W + 2
NG2 = T2 // 6


def _compiler_params():
    cp = pltpu.CompilerParams()
    if "needs_layout_passes" in pltpu.CompilerParams.__dataclass_fields__:
        cp = dataclasses.replace(cp, needs_layout_passes=False)
    return cp


def _sc_bucket(edge_val, edge_u, edge_i):
    mesh = plsc.VectorSubcoreMesh(core_axis_name="c", subcore_axis_name="s")
    out_type = [
        jax.ShapeDtypeStruct((NREG * CAP,), jnp.int32),
        jax.ShapeDtypeStruct((NREG * CAP,), jnp.int32),
        jax.ShapeDtypeStruct((NREG * CAP,), jnp.float32),
        jax.ShapeDtypeStruct((NREG * 16,), jnp.int32),
    ]

    @functools.partial(
        pl.kernel,
        mesh=mesh,
        out_type=out_type,
        compiler_params=_compiler_params(),
        scratch_types=(
            [pltpu.VMEM((W,), jnp.int32) for _ in range(2)]
            + [pltpu.VMEM((W,), jnp.int32) for _ in range(2)]
            + [pltpu.VMEM((W,), jnp.float32) for _ in range(2)]
            + [pltpu.VMEM((192,), jnp.int32) for _ in range(N_CHUNKS)]
            + [pltpu.VMEM((192,), jnp.int32) for _ in range(N_CHUNKS)]
            + [pltpu.VMEM((192,), jnp.float32) for _ in range(N_CHUNKS)]
            + [pltpu.VMEM((16,), jnp.int32)]
            + [pltpu.SemaphoreType.DMA for _ in range(2)]
        ),
    )
    def bucket_kernel(ev_hbm, eu_hbm, ei_hbm, g_scr, l_scr, w_scr, cnts,
                      d0, d1, g0, g1, w0, w1,
                      bg0, bg1, bg2, bg3, bl0, bl1, bl2, bl3,
                      bw0, bw1, bw2, bw3, cntv, si0, si1):
        core = lax.axis_index("c")
        s = lax.axis_index("s")
        st = core * N_TILES + s
        iota16 = jnp.arange(16, dtype=jnp.int32)
        zero16i = jnp.zeros((16,), jnp.int32)
        zero16f = jnp.zeros((16,), jnp.float32)
        I_d, I_g, I_w = (d0, d1), (g0, g1), (w0, w1)
        SBg, SBl, SBw = (bg0, bg1, bg2, bg3), (bl0, bl1, bl2, bl3), \
            (bw0, bw1, bw2, bw3)
        S_i = (si0, si1)

        for d in range(2):
            dest_hbm = eu_hbm if d == 0 else ei_hbm
            src_hbm = ei_hbm if d == 0 else eu_hbm

            def pass_body(b, _, dest_hbm=dest_hbm, src_hbm=src_hbm, d=d):
                def idx_copies(k, slot):
                    off = (st + k * 32) * W
                    return (
                        pltpu.make_async_copy(
                            dest_hbm.at[b, pl.ds(off, W)], I_d[slot],
                            S_i[slot]),
                        pltpu.make_async_copy(
                            src_hbm.at[b, pl.ds(off, W)], I_g[slot],
                            S_i[slot]),
                        pltpu.make_async_copy(
                            ev_hbm.at[b, pl.ds(off, W)], I_w[slot],
                            S_i[slot]),
                    )

                for c in idx_copies(0, 0):
                    c.start()

                def win_body(i, carry):
                    fs = list(carry[:N_CHUNKS])
                    ws = list(carry[N_CHUNKS:])
                    for par in range(2):
                        k = i * 2 + par
                        slot, nslot = par, 1 - par
                        valid = (k < K1) & (st + k * 32 < N_WIN)
                        nvalid = (k + 1 < K1) & (st + (k + 1) * 32 < N_WIN)

                        @pl.when(nvalid)
                        def _():
                            for c in idx_copies(k + 1, nslot):
                                c.start()

                        @pl.when(valid)
                        def _():
                            for c in idx_copies(k, slot):
                                c.wait()

                        vsplat = jnp.where(
                            jnp.full((16,), valid), 1, 0) > 0
                        for g in range(W // 16):
                            sl = pl.ds(16 * g, 16)
                            dv = I_d[slot][sl]
                            gv = I_g[slot][sl]
                            wvv = I_w[slot][sl]
                            for c in range(N_CHUNKS):
                                loc = dv - c * CHUNK
                                m = (loc >= 0) & (loc < CHUNK) & vsplat
                                mi = jnp.where(m, 1, 0)
                                pos = plsc.cumsum(mi)
                                tot = jnp.max(pos)
                                posv = jnp.minimum(fs[c] + pos - 1, 191)
                                plsc.store_scatter(SBg[c], [posv], gv, mask=m)
                                plsc.store_scatter(SBl[c], [posv], loc, mask=m)
                                plsc.store_scatter(SBw[c], [posv], wvv, mask=m)
                                fs[c] = fs[c] + tot
                        for c in range(N_CHUNKS):
                            do = fs[c] >= 128
                            r = ((d * 3 + b) * 32 + st) * N_CHUNKS + c

                            @pl.when(do)
                            def _(c=c, r=r, wr=ws[c]):
                                pltpu.sync_copy(
                                    SBg[c].at[pl.ds(0, 128)],
                                    g_scr.at[pl.ds(pl.multiple_of(r * CAP + wr, 128), 128)])
                                pltpu.sync_copy(
                                    SBl[c].at[pl.ds(0, 128)],
                                    l_scr.at[pl.ds(pl.multiple_of(r * CAP + wr, 128), 128)])
                                pltpu.sync_copy(
                                    SBw[c].at[pl.ds(0, 128)],
                                    w_scr.at[pl.ds(pl.multiple_of(r * CAP + wr, 128), 128)])
                                for g in range(4):
                                    slo = pl.ds(16 * g, 16)
                                    shi = pl.ds(128 + 16 * g, 16)
                                    SBg[c][slo] = SBg[c][shi]
                                    SBl[c][slo] = SBl[c][shi]
                                    SBw[c][slo] = SBw[c][shi]

                            doi = jnp.where(do, 1, 0)
                            fs[c] = fs[c] - 128 * doi
                            ws[c] = ws[c] + 128 * doi
                    return tuple(fs) + tuple(ws)

                carry = lax.fori_loop(
                    0, K1H, win_body, (0,) * (2 * N_CHUNKS))
                for c in range(N_CHUNKS):
                    f_c = carry[c]
                    wr_c = carry[N_CHUNKS + c]
                    fp = (f_c + 63) & (-64)
                    for g in range(4):
                        posv = f_c + iota16 + 16 * g
                        plsc.store_scatter(SBg[c], [posv], zero16i)
                        plsc.store_scatter(SBl[c], [posv], zero16i)
                        plsc.store_scatter(SBw[c], [posv], zero16f)
                    r = ((d * 3 + b) * 32 + st) * N_CHUNKS + c

                    @pl.when(fp == 128)
                    def _(c=c, r=r, wr=wr_c):
                        pltpu.sync_copy(SBg[c].at[pl.ds(0, 128)],
                                        g_scr.at[pl.ds(pl.multiple_of(r * CAP + wr, 128), 128)])
                        pltpu.sync_copy(SBl[c].at[pl.ds(0, 128)],
                                        l_scr.at[pl.ds(pl.multiple_of(r * CAP + wr, 128), 128)])
                        pltpu.sync_copy(SBw[c].at[pl.ds(0, 128)],
                                        w_scr.at[pl.ds(pl.multiple_of(r * CAP + wr, 128), 128)])

                    @pl.when(fp == 64)
                    def _(c=c, r=r, wr=wr_c):
                        pltpu.sync_copy(SBg[c].at[pl.ds(0, 64)],
                                        g_scr.at[pl.ds(pl.multiple_of(r * CAP + wr, 64), 64)])
                        pltpu.sync_copy(SBl[c].at[pl.ds(0, 64)],
                                        l_scr.at[pl.ds(pl.multiple_of(r * CAP + wr, 64), 64)])
                        pltpu.sync_copy(SBw[c].at[pl.ds(0, 64)],
                                        w_scr.at[pl.ds(pl.multiple_of(r * CAP + wr, 64), 64)])

                    total = wr_c + fp
                    for g in range(1):
                        cntv[pl.ds(0, 16)] = jnp.full((16,), total,
                                                      jnp.int32)
                    pltpu.sync_copy(cntv, cnts.at[pl.ds(pl.multiple_of(r * 16, 16), 16)])
                return 0

            lax.fori_loop(0, N_BEH, pass_body, 0)

    return bucket_kernel(edge_val, edge_u, edge_i)


def _sc_accumulate(user_dup, item_dup, zeros, g_scr, l_scr, w_scr, cnts):
    mesh = plsc.VectorSubcoreMesh(core_axis_name="c", subcore_axis_name="s")
    out_type = [
        jax.ShapeDtypeStruct((N_BEH, PAIR_ROWS, 128), jnp.float32),
        jax.ShapeDtypeStruct((N_BEH, PAIR_ROWS, 128), jnp.float32),
    ]

    @functools.partial(
        pl.kernel,
        mesh=mesh,
        out_type=out_type,
        compiler_params=_compiler_params(),
        scratch_types=(
            [pltpu.VMEM((W,), jnp.int32) for _ in range(3)]
            + [pltpu.VMEM((W,), jnp.int32) for _ in range(3)]
            + [pltpu.VMEM((W,), jnp.float32) for _ in range(3)]
            + [pltpu.VMEM((W // 2,), jnp.int32) for _ in range(4)]
            + [pltpu.VMEM((W,), jnp.int32) for _ in range(2)]
            + [pltpu.VMEM((W, 128), jnp.float32) for _ in range(2)]
            + [pltpu.VMEM((W // 2, 128), jnp.float32) for _ in range(2)]
            + [pltpu.VMEM((16,), jnp.int32)]
            + [pltpu.VMEM_SHARED((ACC_P, 128), jnp.float32)]
            + [pltpu.SemaphoreType.DMA for _ in range(3)]
            + [pltpu.SemaphoreType.DMA for _ in range(2)]
            + [pltpu.SemaphoreType.DMA for _ in range(2)]
        ),
    )
    def seg_kernel(ud_hbm, id_hbm, z_hbm, g_scr_h, l_scr_h, w_scr_h, cnts_h,
                   out_u, out_i,
                   d0, d1, d2, g0, g1, g2, w0, w1, w2, la0, la1, lb0, lb1,
                   h0, h1, r0b, r1b, sca, scb, cntv, acc,
                   si0, si1, si2, sr0, sr1, ssa, ssb):
        core = lax.axis_index("c")
        s = lax.axis_index("s")
        zero16 = jnp.zeros((16,), jnp.float32)
        I_l, I_g, I_w = (d0, d1, d2), (g0, g1, g2), (w0, w1, w2)
        M_la, M_lb, M_h = (la0, la1), (lb0, lb1), (h0, h1)
        R = (r0b, r1b)
        S_i = (si0, si1, si2)
        S_r = (sr0, sr1)

        for d in range(2):
            table = id_hbm if d == 0 else ud_hbm
            out_hbm = out_u if d == 0 else out_i

            def pass_body(q, _, table=table, out_hbm=out_hbm, d=d):
                b = q // 2
                cpass = q % 2
                chunk = cpass * 2 + core
                reg0 = ((d * 3 + b) * 32 + 2 * s) * N_CHUNKS + chunk
                reg1 = reg0 + N_CHUNKS

                pltpu.sync_copy(cnts_h.at[pl.ds(pl.multiple_of(reg0 * 16, 16), 16)], cntv)
                nw0 = lax.shift_right_logical(jnp.max(cntv[pl.ds(0, 16)]), 6)
                pltpu.sync_copy(cnts_h.at[pl.ds(pl.multiple_of(reg1 * 16, 16), 16)], cntv)
                nw1 = lax.shift_right_logical(jnp.max(cntv[pl.ds(0, 16)]), 6)
                nw = nw0 + nw1

                z0 = s * ZROWS
                pltpu.sync_copy(z_hbm.at[pl.ds(z0, ZROWS)],
                                acc.at[pl.ds(z0, ZROWS)])
                plsc.subcore_barrier()

                def idx_copies(n, slot):
                    rsel = jnp.where(n < nw0, reg0, reg1)
                    woff = jnp.where(n < nw0, n, n - nw0) * W
                    base = pl.multiple_of(rsel * CAP + woff, 64)
                    return (
                        pltpu.make_async_copy(
                            l_scr_h.at[pl.ds(base, W)], I_l[slot], S_i[slot]),
                        pltpu.make_async_copy(
                            g_scr_h.at[pl.ds(base, W)], I_g[slot], S_i[slot]),
                        pltpu.make_async_copy(
                            w_scr_h.at[pl.ds(base, W)], I_w[slot], S_i[slot]),
                    )

                def gather_copy(slot3, slot2):
                    return pltpu.make_async_copy(
                        table.at[I_g[slot3]], R[slot2], S_r[slot2])

                @pl.when(nw > 0)
                def _():
                    for c in idx_copies(0, 0):
                        c.start()

                @pl.loop(0, NG2)
                def _(gg):
                    for p in range(6):
                        n = gg * 6 + p
                        i3, nxt3, prev3 = p % 3, (p + 1) % 3, (p + 2) % 3
                        r2, prev2 = p % 2, (p + 1) % 2

                        @pl.when(n + 1 < nw)
                        def _():
                            for c in idx_copies(n + 1, nxt3):
                                c.start()

                        @pl.when(n < nw)
                        def _():
                            @pl.when(n >= 2)
                            def _():
                                pltpu.make_async_copy(
                                    sca, acc.at[M_la[r2]], ssa).wait()
                                pltpu.make_async_copy(
                                    scb, acc.at[M_lb[r2]], ssb).wait()

                            for c in idx_copies(n, i3):
                                c.wait()
                            gather_copy(i3, r2).start()
                            for g in range(W // 16):
                                sl = pl.ds(16 * g, 16)
                                loc = I_l[i3][sl]
                                lp = lax.shift_right_logical(loc, 1)
                                if g < 2:
                                    M_la[r2][sl] = lp
                                else:
                                    M_lb[r2][pl.ds(16 * (g - 2), 16)] = lp
                                M_h[r2][sl] = lax.shift_left(loc & 1, 6)

                        @pl.when((n >= 1) & (n - 1 < nw))
                        def _():
                            gather_copy(prev3, prev2).wait()

                            def scale_rows(lo, dst):
                                @pl.loop(lo, lo + W // 2, step=4)
                                def _(j0):
                                    for u in range(4):
                                        j = j0 + u
                                        jj = jnp.full((16,), j, jnp.int32)
                                        wspl = plsc.load_gather(
                                            I_w[prev3], [jj])
                                        dbs = plsc.load_gather(
                                            M_h[prev2], [jj])
                                        f0 = jnp.where(dbs == 0, wspl, zero16)
                                        f1 = wspl - f0
                                        for qq in range(4):
                                            v = R[prev2][j, pl.ds(16 * qq, 16)]
                                            dj = j - lo
                                            dst[dj, pl.ds(16 * qq, 16)] = (
                                                v * f0)
                                            dst[dj, pl.ds(64 + 16 * qq, 16)] = (
                                                v * f1)

                            scale_rows(0, sca)
                            pltpu.make_async_copy(
                                sca, acc.at[M_la[prev2]], ssa).start(add=True)
                            scale_rows(W // 2, scb)
                            pltpu.make_async_copy(
                                scb, acc.at[M_lb[prev2]], ssb).start(add=True)

                @pl.when(nw >= 1)
                def _():
                    pltpu.make_async_copy(sca, acc.at[M_la[0]], ssa).wait()
                    pltpu.make_async_copy(scb, acc.at[M_lb[0]], ssb).wait()

                @pl.when(nw >= 2)
                def _():
                    pltpu.make_async_copy(sca, acc.at[M_la[0]], ssa).wait()
                    pltpu.make_async_copy(scb, acc.at[M_lb[0]], ssb).wait()

                plsc.subcore_barrier()
                f0r = s * FROWS
                pltpu.sync_copy(
                    acc.at[pl.ds(f0r, FROWS)],
                    out_hbm.at[b, pl.ds(chunk * CHUNK_P + f0r, FROWS)])
                plsc.subcore_barrier()
                return 0

            lax.fori_loop(0, N_BEH * 2, pass_body, 0)

    return seg_kernel(user_dup, item_dup, zeros, g_scr, l_scr, w_scr, cnts)


BLK = 2048


def _proj_body(x_ref, w_ref, stack_ref, mean_ref):
    acc = jnp.zeros((BLK, DIM), jnp.float32)
    for b in range(N_BEH):
        z = jnp.dot(x_ref[b], w_ref[...], preferred_element_type=jnp.float32)
        stack_ref[b] = jax.nn.sigmoid(z)
        acc = acc + z
    mean_ref[...] = jax.nn.sigmoid(acc * (1.0 / N_BEH))


def _project(stack, weight):
    grid = (PAD_ROWS // BLK,)
    return pl.pallas_call(
        _proj_body,
        grid=grid,
        in_specs=[
            pl.BlockSpec((N_BEH, BLK, DIM), lambda i: (0, i, 0)),
            pl.BlockSpec((DIM, DIM), lambda i: (0, 0)),
        ],
        out_specs=[
            pl.BlockSpec((N_BEH, BLK, DIM), lambda i: (0, i, 0)),
            pl.BlockSpec((BLK, DIM), lambda i: (i, 0)),
        ],
        out_shape=[
            jax.ShapeDtypeStruct((N_BEH, PAD_ROWS, DIM), jnp.float32),
            jax.ShapeDtypeStruct((PAD_ROWS, DIM), jnp.float32),
        ],
    )(stack, weight)


def kernel(user_embedding, item_embedding, u_w, i_w, edge_val, edge_u, edge_i):
    user_dup = jnp.concatenate([user_embedding, user_embedding], axis=1)
    item_dup = jnp.concatenate([item_embedding, item_embedding], axis=1)
    zeros = jnp.zeros((ACC_P, 128), jnp.float32)
    g_scr, l_scr, w_scr, cnts = _sc_bucket(edge_val, edge_u, edge_i)
    ue_p, ie_p = _sc_accumulate(
        user_dup, item_dup, zeros, g_scr, l_scr, w_scr, cnts)
    ue_stack = ue_p.reshape(N_BEH, PAD_ROWS, DIM)
    ie_stack = ie_p.reshape(N_BEH, PAD_ROWS, DIM)
    us_out, u_mean = _project(ue_stack, u_w)
    is_out, i_mean = _project(ie_stack, i_w)
    return (
        u_mean[:N_ROWS],
        i_mean[:N_ROWS],
        us_out[:, :N_ROWS],
        is_out[:, :N_ROWS],
    )


# speedup vs baseline: 8.0838x; 1.1936x over previous
import dataclasses
import functools

import jax
import jax.numpy as jnp
from jax import lax
from jax.experimental import pallas as pl
from jax.experimental.pallas import tpu as pltpu
from jax.experimental.pallas import tpu_sc as plsc

N_ROWS = 100000
DIM = 64
N_BEH = 3
N_EDGES = 1000000

CHUNK = 25600
N_CHUNKS = 4
PAD_ROWS = CHUNK * N_CHUNKS
PAIR_ROWS = PAD_ROWS // 2
CHUNK_P = CHUNK // 2
ACC_P = CHUNK_P + 128
W = 64
N_WIN = N_EDGES // W
N_TILES = 16
ZROWS = ACC_P // N_TILES
FROWS = CHUNK_P // N_TILES

CAP = 8960
NREG = 2 * N_BEH * 32 * N_CHUNKS
K1 = (N_WIN + 31) // 32
K1H = (K1 + 1) // 2
T2 = 2 * CAP // W + 2
NG2 = T2 // 6


def _compiler_params(tc_tiling=True):
    cp = pltpu.CompilerParams()
    if "needs_layout_passes" in pltpu.CompilerParams.__dataclass_fields__:
        cp = dataclasses.replace(cp, needs_layout_passes=False)
    if not tc_tiling:
        cp = dataclasses.replace(cp, use_tc_tiling_on_sc=False)
    return cp


def _sc_bucket(edge_val, edge_u, edge_i):
    mesh = plsc.VectorSubcoreMesh(core_axis_name="c", subcore_axis_name="s")
    out_type = [
        jax.ShapeDtypeStruct((NREG * CAP,), jnp.int32),
        jax.ShapeDtypeStruct((NREG * CAP,), jnp.int32),
        jax.ShapeDtypeStruct((NREG * CAP,), jnp.float32),
        jax.ShapeDtypeStruct((NREG * 16,), jnp.int32),
    ]

    @functools.partial(
        pl.kernel,
        mesh=mesh,
        out_type=out_type,
        compiler_params=_compiler_params(),
        scratch_types=(
            [pltpu.VMEM((W,), jnp.int32) for _ in range(2)]
            + [pltpu.VMEM((W,), jnp.int32) for _ in range(2)]
            + [pltpu.VMEM((W,), jnp.float32) for _ in range(2)]
            + [pltpu.VMEM((192,), jnp.int32) for _ in range(N_CHUNKS)]
            + [pltpu.VMEM((192,), jnp.int32) for _ in range(N_CHUNKS)]
            + [pltpu.VMEM((192,), jnp.float32) for _ in range(N_CHUNKS)]
            + [pltpu.VMEM((16,), jnp.int32)]
            + [pltpu.SemaphoreType.DMA for _ in range(2)]
        ),
    )
    def bucket_kernel(ev_hbm, eu_hbm, ei_hbm, g_scr, l_scr, w_scr, cnts,
                      d0, d1, g0, g1, w0, w1,
                      bg0, bg1, bg2, bg3, bl0, bl1, bl2, bl3,
                      bw0, bw1, bw2, bw3, cntv, si0, si1):
        core = lax.axis_index("c")
        s = lax.axis_index("s")
        st = core * N_TILES + s
        iota16 = jnp.arange(16, dtype=jnp.int32)
        zero16i = jnp.zeros((16,), jnp.int32)
        zero16f = jnp.zeros((16,), jnp.float32)
        I_d, I_g, I_w = (d0, d1), (g0, g1), (w0, w1)
        SBg, SBl, SBw = (bg0, bg1, bg2, bg3), (bl0, bl1, bl2, bl3), \
            (bw0, bw1, bw2, bw3)
        S_i = (si0, si1)

        for d in range(2):
            dest_hbm = eu_hbm if d == 0 else ei_hbm
            src_hbm = ei_hbm if d == 0 else eu_hbm

            def pass_body(b, _, dest_hbm=dest_hbm, src_hbm=src_hbm, d=d):
                def idx_copies(k, slot):
                    off = (st + k * 32) * W
                    return (
                        pltpu.make_async_copy(
                            dest_hbm.at[b, pl.ds(off, W)], I_d[slot],
                            S_i[slot]),
                        pltpu.make_async_copy(
                            src_hbm.at[b, pl.ds(off, W)], I_g[slot],
                            S_i[slot]),
                        pltpu.make_async_copy(
                            ev_hbm.at[b, pl.ds(off, W)], I_w[slot],
                            S_i[slot]),
                    )

                for c in idx_copies(0, 0):
                    c.start()

                def win_body(i, carry):
                    fs = list(carry[:N_CHUNKS])
                    ws = list(carry[N_CHUNKS:])
                    for par in range(2):
                        k = i * 2 + par
                        slot, nslot = par, 1 - par
                        valid = (k < K1) & (st + k * 32 < N_WIN)
                        nvalid = (k + 1 < K1) & (st + (k + 1) * 32 < N_WIN)

                        @pl.when(nvalid)
                        def _():
                            for c in idx_copies(k + 1, nslot):
                                c.start()

                        @pl.when(valid)
                        def _():
                            for c in idx_copies(k, slot):
                                c.wait()

                        vsplat = jnp.where(
                            jnp.full((16,), valid), 1, 0) > 0
                        for g in range(W // 16):
                            sl = pl.ds(16 * g, 16)
                            dv = I_d[slot][sl]
                            gv = I_g[slot][sl]
                            wvv = I_w[slot][sl]
                            for c in range(N_CHUNKS):
                                loc = dv - c * CHUNK
                                m = (loc >= 0) & (loc < CHUNK) & vsplat
                                mi = jnp.where(m, 1, 0)
                                pos = plsc.cumsum(mi)
                                tot = jnp.max(pos)
                                posv = jnp.minimum(fs[c] + pos - 1, 191)
                                plsc.store_scatter(SBg[c], [posv], gv, mask=m)
                                plsc.store_scatter(SBl[c], [posv], loc, mask=m)
                                plsc.store_scatter(SBw[c], [posv], wvv, mask=m)
                                fs[c] = fs[c] + tot
                        for c in range(N_CHUNKS):
                            do = fs[c] >= 128
                            r = ((d * 3 + b) * 32 + st) * N_CHUNKS + c

                            @pl.when(do)
                            def _(c=c, r=r, wr=ws[c]):
                                pltpu.sync_copy(
                                    SBg[c].at[pl.ds(0, 128)],
                                    g_scr.at[pl.ds(pl.multiple_of(r * CAP + wr, 128), 128)])
                                pltpu.sync_copy(
                                    SBl[c].at[pl.ds(0, 128)],
                                    l_scr.at[pl.ds(pl.multiple_of(r * CAP + wr, 128), 128)])
                                pltpu.sync_copy(
                                    SBw[c].at[pl.ds(0, 128)],
                                    w_scr.at[pl.ds(pl.multiple_of(r * CAP + wr, 128), 128)])
                                for g in range(4):
                                    slo = pl.ds(16 * g, 16)
                                    shi = pl.ds(128 + 16 * g, 16)
                                    SBg[c][slo] = SBg[c][shi]
                                    SBl[c][slo] = SBl[c][shi]
                                    SBw[c][slo] = SBw[c][shi]

                            doi = jnp.where(do, 1, 0)
                            fs[c] = fs[c] - 128 * doi
                            ws[c] = ws[c] + 128 * doi
                    return tuple(fs) + tuple(ws)

                carry = lax.fori_loop(
                    0, K1H, win_body, (0,) * (2 * N_CHUNKS))
                for c in range(N_CHUNKS):
                    f_c = carry[c]
                    wr_c = carry[N_CHUNKS + c]
                    fp = (f_c + 63) & (-64)
                    for g in range(4):
                        posv = f_c + iota16 + 16 * g
                        plsc.store_scatter(SBg[c], [posv], zero16i)
                        plsc.store_scatter(SBl[c], [posv], zero16i)
                        plsc.store_scatter(SBw[c], [posv], zero16f)
                    r = ((d * 3 + b) * 32 + st) * N_CHUNKS + c

                    @pl.when(fp == 128)
                    def _(c=c, r=r, wr=wr_c):
                        pltpu.sync_copy(SBg[c].at[pl.ds(0, 128)],
                                        g_scr.at[pl.ds(pl.multiple_of(r * CAP + wr, 128), 128)])
                        pltpu.sync_copy(SBl[c].at[pl.ds(0, 128)],
                                        l_scr.at[pl.ds(pl.multiple_of(r * CAP + wr, 128), 128)])
                        pltpu.sync_copy(SBw[c].at[pl.ds(0, 128)],
                                        w_scr.at[pl.ds(pl.multiple_of(r * CAP + wr, 128), 128)])

                    @pl.when(fp == 64)
                    def _(c=c, r=r, wr=wr_c):
                        pltpu.sync_copy(SBg[c].at[pl.ds(0, 64)],
                                        g_scr.at[pl.ds(pl.multiple_of(r * CAP + wr, 64), 64)])
                        pltpu.sync_copy(SBl[c].at[pl.ds(0, 64)],
                                        l_scr.at[pl.ds(pl.multiple_of(r * CAP + wr, 64), 64)])
                        pltpu.sync_copy(SBw[c].at[pl.ds(0, 64)],
                                        w_scr.at[pl.ds(pl.multiple_of(r * CAP + wr, 64), 64)])

                    total = wr_c + fp
                    for g in range(1):
                        cntv[pl.ds(0, 16)] = jnp.full((16,), total,
                                                      jnp.int32)
                    pltpu.sync_copy(cntv, cnts.at[pl.ds(pl.multiple_of(r * 16, 16), 16)])
                return 0

            lax.fori_loop(0, N_BEH, pass_body, 0)

    return bucket_kernel(edge_val, edge_u, edge_i)


def _sc_accumulate(user_dup, item_dup, zeros, g_scr, l_scr, w_scr, cnts):
    mesh = plsc.VectorSubcoreMesh(core_axis_name="c", subcore_axis_name="s")
    out_type = [
        jax.ShapeDtypeStruct((N_BEH, PAD_ROWS, DIM), jnp.float32),
        jax.ShapeDtypeStruct((N_BEH, PAD_ROWS, DIM), jnp.float32),
    ]

    @functools.partial(
        pl.kernel,
        mesh=mesh,
        out_type=out_type,
        compiler_params=_compiler_params(tc_tiling=False),
        scratch_types=(
            [pltpu.VMEM((W,), jnp.int32) for _ in range(3)]
            + [pltpu.VMEM((W,), jnp.int32) for _ in range(3)]
            + [pltpu.VMEM((W,), jnp.float32) for _ in range(3)]
            + [pltpu.VMEM((W // 2,), jnp.int32) for _ in range(4)]
            + [pltpu.VMEM((W, DIM), jnp.float32) for _ in range(2)]
            + [pltpu.VMEM((W // 2, DIM), jnp.float32) for _ in range(2)]
            + [pltpu.VMEM((16,), jnp.int32)]
            + [pltpu.VMEM_SHARED((CHUNK, DIM), jnp.float32)]
            + [pltpu.SemaphoreType.DMA for _ in range(3)]
            + [pltpu.SemaphoreType.DMA for _ in range(2)]
            + [pltpu.SemaphoreType.DMA for _ in range(2)]
        ),
    )
    def seg_kernel(ud_hbm, id_hbm, z_hbm, g_scr_h, l_scr_h, w_scr_h, cnts_h,
                   out_u, out_i,
                   d0, d1, d2, g0, g1, g2, w0, w1, w2, la0, la1, lb0, lb1,
                   r0b, r1b, sca, scb, cntv, acc,
                   si0, si1, si2, sr0, sr1, ssa, ssb):
        core = lax.axis_index("c")
        s = lax.axis_index("s")
        zero16 = jnp.zeros((16,), jnp.float32)
        I_l, I_g, I_w = (d0, d1, d2), (g0, g1, g2), (w0, w1, w2)
        M_la, M_lb = (la0, la1), (lb0, lb1)
        R = (r0b, r1b)
        S_i = (si0, si1, si2)
        S_r = (sr0, sr1)

        for d in range(2):
            table = id_hbm if d == 0 else ud_hbm
            out_hbm = out_u if d == 0 else out_i

            def pass_body(q, _, table=table, out_hbm=out_hbm, d=d):
                b = q // 2
                cpass = q % 2
                chunk = cpass * 2 + core
                reg0 = ((d * 3 + b) * 32 + 2 * s) * N_CHUNKS + chunk
                reg1 = reg0 + N_CHUNKS

                pltpu.sync_copy(cnts_h.at[pl.ds(pl.multiple_of(reg0 * 16, 16), 16)], cntv)
                nw0 = lax.shift_right_logical(jnp.max(cntv[pl.ds(0, 16)]), 6)
                pltpu.sync_copy(cnts_h.at[pl.ds(pl.multiple_of(reg1 * 16, 16), 16)], cntv)
                nw1 = lax.shift_right_logical(jnp.max(cntv[pl.ds(0, 16)]), 6)
                nw = nw0 + nw1

                z0 = s * (CHUNK // N_TILES)
                pltpu.sync_copy(z_hbm.at[pl.ds(z0, CHUNK // N_TILES)],
                                acc.at[pl.ds(z0, CHUNK // N_TILES)])
                plsc.subcore_barrier()

                def idx_copies(n, slot):
                    rsel = jnp.where(n < nw0, reg0, reg1)
                    woff = jnp.where(n < nw0, n, n - nw0) * W
                    base = pl.multiple_of(rsel * CAP + woff, 64)
                    return (
                        pltpu.make_async_copy(
                            l_scr_h.at[pl.ds(base, W)], I_l[slot], S_i[slot]),
                        pltpu.make_async_copy(
                            g_scr_h.at[pl.ds(base, W)], I_g[slot], S_i[slot]),
                        pltpu.make_async_copy(
                            w_scr_h.at[pl.ds(base, W)], I_w[slot], S_i[slot]),
                    )

                def gather_copy(slot3, slot2):
                    return pltpu.make_async_copy(
                        table.at[I_g[slot3]], R[slot2], S_r[slot2])

                @pl.when(nw > 0)
                def _():
                    for c in idx_copies(0, 0):
                        c.start()

                @pl.loop(0, NG2)
                def _(gg):
                    for p in range(6):
                        n = gg * 6 + p
                        i3, nxt3, prev3 = p % 3, (p + 1) % 3, (p + 2) % 3
                        r2, prev2 = p % 2, (p + 1) % 2

                        @pl.when(n + 1 < nw)
                        def _():
                            for c in idx_copies(n + 1, nxt3):
                                c.start()

                        @pl.when(n < nw)
                        def _():
                            @pl.when(n >= 2)
                            def _():
                                pltpu.make_async_copy(
                                    sca, acc.at[M_la[r2]], ssa).wait()
                                pltpu.make_async_copy(
                                    scb, acc.at[M_lb[r2]], ssb).wait()

                            for c in idx_copies(n, i3):
                                c.wait()
                            gather_copy(i3, r2).start()
                            for g in range(W // 16):
                                sl = pl.ds(16 * g, 16)
                                loc = I_l[i3][sl]
                                if g < 2:
                                    M_la[r2][sl] = loc
                                else:
                                    M_lb[r2][pl.ds(16 * (g - 2), 16)] = loc

                        @pl.when((n >= 1) & (n - 1 < nw))
                        def _():
                            gather_copy(prev3, prev2).wait()

                            def scale_rows(lo, dst):
                                @pl.loop(lo, lo + W // 2, step=4)
                                def _(j0):
                                    for u in range(4):
                                        j = j0 + u
                                        jj = jnp.full((16,), j, jnp.int32)
                                        wspl = plsc.load_gather(
                                            I_w[prev3], [jj])
                                        for qq in range(4):
                                            v = R[prev2][j, pl.ds(16 * qq, 16)]
                                            dst[j - lo, pl.ds(16 * qq, 16)] = (
                                                v * wspl)

                            scale_rows(0, sca)
                            pltpu.make_async_copy(
                                sca, acc.at[M_la[prev2]], ssa).start(add=True)
                            scale_rows(W // 2, scb)
                            pltpu.make_async_copy(
                                scb, acc.at[M_lb[prev2]], ssb).start(add=True)

                @pl.when(nw >= 1)
                def _():
                    pltpu.make_async_copy(sca, acc.at[M_la[0]], ssa).wait()
                    pltpu.make_async_copy(scb, acc.at[M_lb[0]], ssb).wait()

                @pl.when(nw >= 2)
                def _():
                    pltpu.make_async_copy(sca, acc.at[M_la[0]], ssa).wait()
                    pltpu.make_async_copy(scb, acc.at[M_lb[0]], ssb).wait()

                plsc.subcore_barrier()
                f0r = s * (CHUNK // N_TILES)
                pltpu.sync_copy(
                    acc.at[pl.ds(f0r, CHUNK // N_TILES)],
                    out_hbm.at[b, pl.ds(chunk * CHUNK + f0r,
                                        CHUNK // N_TILES)])
                plsc.subcore_barrier()
                return 0

            lax.fori_loop(0, N_BEH * 2, pass_body, 0)

    return seg_kernel(user_dup, item_dup, zeros, g_scr, l_scr, w_scr, cnts)


BLK = 2048


def _proj_body(x_ref, w_ref, stack_ref, mean_ref):
    acc = jnp.zeros((BLK, DIM), jnp.float32)
    for b in range(N_BEH):
        z = jnp.dot(x_ref[b], w_ref[...], preferred_element_type=jnp.float32)
        stack_ref[b] = jax.nn.sigmoid(z)
        acc = acc + z
    mean_ref[...] = jax.nn.sigmoid(acc * (1.0 / N_BEH))


def _project(stack, weight):
    grid = (PAD_ROWS // BLK,)
    return pl.pallas_call(
        _proj_body,
        grid=grid,
        in_specs=[
            pl.BlockSpec((N_BEH, BLK, DIM), lambda i: (0, i, 0)),
            pl.BlockSpec((DIM, DIM), lambda i: (0, 0)),
        ],
        out_specs=[
            pl.BlockSpec((N_BEH, BLK, DIM), lambda i: (0, i, 0)),
            pl.BlockSpec((BLK, DIM), lambda i: (i, 0)),
        ],
        out_shape=[
            jax.ShapeDtypeStruct((N_BEH, PAD_ROWS, DIM), jnp.float32),
            jax.ShapeDtypeStruct((PAD_ROWS, DIM), jnp.float32),
        ],
    )(stack, weight)


def kernel(user_embedding, item_embedding, u_w, i_w, edge_val, edge_u, edge_i):
    zeros = jnp.zeros((CHUNK, DIM), jnp.float32)
    g_scr, l_scr, w_scr, cnts = _sc_bucket(edge_val, edge_u, edge_i)
    ue_stack, ie_stack = _sc_accumulate(
        user_embedding, item_embedding, zeros, g_scr, l_scr, w_scr, cnts)
    us_out, u_mean = _project(ue_stack, u_w)
    is_out, i_mean = _project(ie_stack, i_w)
    return (
        u_mean[:N_ROWS],
        i_mean[:N_ROWS],
        us_out[:, :N_ROWS],
        is_out[:, :N_ROWS],
    )
